# baseline (speedup 1.0000x reference)
import functools

import jax
import jax.numpy as jnp
from jax import lax
from jax.experimental import pallas as pl
from jax.experimental.pallas import tpu as pltpu
from jax.experimental.pallas import tpu_sc as plsc

DIM = 64
NBUF = 2
CHUNK = 128


def _emb_call(idx, wt128, num_rows):
    info = plsc.get_sparse_core_info()
    nc, ns = info.num_cores, info.num_subcores
    nw = nc * ns
    rows_per_w = num_rows // nw
    n_chunks = rows_per_w // CHUNK
    n_outer = n_chunks // NBUF

    mesh = plsc.VectorSubcoreMesh(core_axis_name="c", subcore_axis_name="s")

    @functools.partial(
        pl.kernel,
        mesh=mesh,
        out_type=jax.ShapeDtypeStruct((num_rows, DIM), jnp.float32),
        scratch_types=[
            pltpu.VMEM((NBUF, CHUNK + 16), jnp.int32),
            pltpu.VMEM((NBUF, CHUNK), jnp.int32),
            pltpu.VMEM((NBUF, CHUNK, 128), jnp.float32),
            pltpu.VMEM((NBUF, CHUNK, DIM), jnp.float32),
            [pltpu.SemaphoreType.DMA] * NBUF,
            [pltpu.SemaphoreType.DMA] * NBUF,
        ],
    )
    def emb(idx_hbm, wt_hbm, out_hbm, idxc, gv, rows_v, outb, gsems, ssems):
        wid = lax.axis_index("s") * nc + lax.axis_index("c")
        base = wid * rows_per_w

        def outer(g, carry):
            for k in range(NBUF):
                i = g * NBUF + k
                off = base + i * CHUNK

                @pl.when(g > 0)
                def _wait_store():
                    off_prev = base + (i - NBUF) * CHUNK
                    pltpu.make_async_copy(
                        outb.at[k], out_hbm.at[pl.ds(off_prev, CHUNK)], ssems[k]
                    ).wait()

                pltpu.sync_copy(
                    idx_hbm.at[pl.ds(off, CHUNK)], idxc.at[k, pl.ds(0, CHUNK)]
                )
                for jb in range(CHUNK // 16):
                    t = idxc[k, pl.ds(jb * 16, 16)]
                    gv[k, pl.ds(jb * 16, 16)] = lax.shift_right_logical(t, 1)
                pltpu.async_copy(wt_hbm.at[gv.at[k]], rows_v.at[k], gsems[k])
            for k in range(NBUF):
                i = g * NBUF + k
                off = base + i * CHUNK
                pltpu.make_async_copy(
                    wt_hbm.at[gv.at[k]], rows_v.at[k], gsems[k]
                ).wait()

                def fix(jb, fcarry):
                    j0 = pl.multiple_of(jb * 16, 16)
                    tv = idxc[k, pl.ds(j0, 16)]
                    for l in range(16):
                        j = j0 + l
                        h = (tv[l] & 1) * DIM
                        for c4 in range(DIM // 16):
                            outb[k, j, pl.ds(c4 * 16, 16)] = rows_v[
                                k, j, pl.ds(pl.multiple_of(h + c4 * 16, 16), 16)
                            ]
                    return fcarry

                lax.fori_loop(0, CHUNK // 16, fix, 0)
                pltpu.async_copy(
                    outb.at[k], out_hbm.at[pl.ds(off, CHUNK)], ssems[k]
                )
            return carry

        lax.fori_loop(0, n_outer, outer, 0)

        for k in range(NBUF):
            i = (n_outer - 1) * NBUF + k
            off = base + i * CHUNK
            pltpu.make_async_copy(
                outb.at[k], out_hbm.at[pl.ds(off, CHUNK)], ssems[k]
            ).wait()

    return emb(idx, wt128)


def kernel(tokens, weight):
    b, s = tokens.shape
    num_rows = b * s
    idx = tokens.reshape(num_rows).astype(jnp.int32)
    wt128 = jnp.reshape(weight, (weight.shape[0] // 2, 128))
    out = _emb_call(idx, wt128, num_rows)
    return out.reshape(b, s, DIM)
